# baseline (speedup 1.0000x reference)
import functools

import jax
import jax.numpy as jnp
from jax import lax
from jax.experimental import pallas as pl
from jax.experimental.pallas import tpu as pltpu
from jax.experimental.pallas import tpu_sc as plsc

_LANES = 16
_CH = 128
_NSLOT = 4
_PAD = 129
_BPAD = 137


@functools.cache
def _build_transpose(vocab, embed):
    info = plsc.get_sparse_core_info()
    nw = info.num_cores * info.num_subcores
    assert embed == 64
    blkw = 2 * _CH
    nfull = vocab // blkw
    tailn = vocab - nfull * blkw
    assert tailn % 2 == 0
    bpw = -(-nfull // nw)
    bpw += bpw % 2
    ecols2 = _CH // _LANES

    mesh = plsc.VectorSubcoreMesh(core_axis_name="c", subcore_axis_name="s")

    @functools.partial(
        pl.kernel,
        out_type=jax.ShapeDtypeStruct((vocab // 2, _CH), jnp.float32),
        mesh=mesh,
        compiler_params=pltpu.CompilerParams(
            use_tc_tiling_on_sc=True, needs_layout_passes=False
        ),
        scratch_types=(
            [pltpu.VMEM((64, 2 * _CH + 9), jnp.float32) for _ in range(2)]
            + [pltpu.VMEM((_CH, _CH), jnp.float32) for _ in range(2)]
            + [pltpu.VMEM((tailn // 2, _CH), jnp.float32)]
            + [pltpu.SemaphoreType.DMA for _ in range(5)]
        ),
    )
    def transpose_kernel(tt_hbm, tail_hbm, tmp_hbm, b0, b1, o0, o1, tl, *sems):
        bb = (b0, b1)
        oo = (o0, o1)
        rsem = sems[0:2]
        wsem = sems[2:4]
        tsem = sems[4]

        wid = lax.axis_index("s") * info.num_cores + lax.axis_index("c")
        t0 = wid * bpw
        tmax = jnp.minimum(t0 + bpw, nfull)

        def fire_read(t, s):
            pltpu.async_copy(
                tt_hbm.at[:, pl.ds(t * blkw, blkw)], bb[s].at[:, pl.ds(0, blkw)],
                rsem[s],
            )

        def wait_read(s):
            pltpu.make_async_copy(
                tt_hbm.at[:, pl.ds(0, blkw)], bb[s].at[:, pl.ds(0, blkw)], rsem[s]
            ).wait()

        def fire_store(t, s):
            pltpu.async_copy(oo[s], tmp_hbm.at[pl.ds(t * _CH, _CH)], wsem[s])

        def wait_store(s):
            pltpu.make_async_copy(oo[s], tmp_hbm.at[pl.ds(0, _CH)], wsem[s]).wait()

        @pl.when(wid == 0)
        def _():
            pltpu.async_copy(tail_hbm, tl, tsem)
            pltpu.make_async_copy(tail_hbm, tl, tsem).wait()
            pltpu.async_copy(tl, tmp_hbm.at[pl.ds(nfull * _CH, tailn // 2)], tsem)
            pltpu.make_async_copy(
                tl, tmp_hbm.at[pl.ds(0, tailn // 2)], tsem
            ).wait()

        @pl.when(t0 < tmax)
        def _():
            fire_read(t0, 0)

        @pl.when(t0 + 1 < tmax)
        def _():
            fire_read(t0 + 1, 1)

        iota = lax.iota(jnp.int32, _LANES)
        erow = [iota + (16 * j) % 64 for j in range(ecols2)]

        def blk_body(k2, carry):
            for s in range(2):
                t = t0 + k2 * 2 + s

                @pl.when(t < tmax)
                def _():
                    wait_read(s)

                    @pl.when(k2 >= 1)
                    def _():
                        wait_store(s)

                    @plsc.parallel_loop(0, _CH, 1, unroll=8)
                    def _(q):
                        cv0 = jnp.full((_LANES,), 2 * q, jnp.int32)
                        cv1 = cv0 + 1
                        for j in range(ecols2):
                            v = plsc.load_gather(
                                bb[s], [erow[j], cv0 if j < 4 else cv1]
                            )
                            oo[s][q, pl.ds(j * _LANES, _LANES)] = v

                    fire_store(t, s)

                    @pl.when(t + 2 < tmax)
                    def _():
                        fire_read(t + 2, s)

            return carry

        lax.fori_loop(0, bpw // 2, blk_body, 0)
        for s in range(2):
            @pl.when(t0 + s < tmax)
            def _():
                wait_store(s)

    return transpose_kernel


@functools.cache
def _build_lookup(batch, seq, embed, vocab):
    info = plsc.get_sparse_core_info()
    nw = info.num_cores * info.num_subcores
    assert batch % nw == 0 and seq == _NSLOT * _CH and embed == 64
    rows_per_w = batch // nw
    nch = rows_per_w * _NSLOT
    st_n = seq // _CH
    ecols = embed // _LANES

    mesh = plsc.VectorSubcoreMesh(core_axis_name="c", subcore_axis_name="s")

    @functools.partial(
        pl.kernel,
        out_type=jax.ShapeDtypeStruct((batch, 8, st_n, 8, _CH), jnp.float32),
        mesh=mesh,
        compiler_params=pltpu.CompilerParams(
            use_tc_tiling_on_sc=False, needs_layout_passes=False
        ),
        scratch_types=(
            [pltpu.VMEM((seq, embed), jnp.float32)]
            + [pltpu.VMEM((_CH,), jnp.int32) for _ in range(_NSLOT)]
            + [pltpu.VMEM((_CH, 64), jnp.float32) for _ in range(_NSLOT)]
            + [pltpu.VMEM((8, 8, _PAD), jnp.float32) for _ in range(_NSLOT)]
            + [pltpu.SemaphoreType.DMA for _ in range(3 * _NSLOT)]
        ),
    )
    def embed_kernel(ids_hbm, tok_hbm, pos_hbm, out_hbm, pos_v, *scratch):
        idx = scratch[:_NSLOT]
        g = scratch[_NSLOT : 2 * _NSLOT]
        ot = scratch[2 * _NSLOT : 3 * _NSLOT]
        sems = scratch[3 * _NSLOT :]
        isem = sems[:_NSLOT]
        gsem = sems[_NSLOT : 2 * _NSLOT]
        osem = sems[2 * _NSLOT : 3 * _NSLOT]

        wid = lax.axis_index("s") * info.num_cores + lax.axis_index("c")
        row0 = wid * rows_per_w
        pltpu.sync_copy(pos_hbm, pos_v)

        def fire_idx(row, c, s):
            pltpu.async_copy(ids_hbm.at[row, pl.ds(c * _CH, _CH)], idx[s], isem[s])

        def wait_idx(s):
            pltpu.make_async_copy(ids_hbm.at[0, pl.ds(0, _CH)], idx[s], isem[s]).wait()

        def fire_gather(s):
            pltpu.async_copy(tok_hbm.at[idx[s]], g[s], gsem[s])

        def wait_gather(s):
            pltpu.make_async_copy(tok_hbm.at[pl.ds(0, _CH)], g[s], gsem[s]).wait()

        def fire_store(row, s):
            pltpu.async_copy(
                ot[s].at[:, :, pl.ds(0, _CH)], out_hbm.at[row, :, s, :, :], osem[s]
            )

        def wait_store(s):
            pltpu.make_async_copy(
                ot[s].at[:, :, pl.ds(0, _CH)], out_hbm.at[0, :, 0, :, :], osem[s]
            ).wait()

        for s in range(3):
            fire_idx(row0, s, s)
        wait_idx(0)
        fire_gather(0)
        wait_idx(1)
        fire_gather(1)

        iota = lax.iota(jnp.int32, _LANES)
        etv = [lax.shift_right_logical(iota + 16 * j, 3) for j in range(ecols)]
        e8v = [lax.bitwise_and(iota + 16 * j, 7) for j in range(ecols)]

        def group_body(kk, carry):
            row = row0 + kk
            for b in range(_NSLOT):
                k = kk * _NSLOT + b
                wait_gather(b)

                s3 = (b + 3) % _NSLOT

                @pl.when(k < nch - 3)
                def _():
                    fire_idx(row + (b + 3) // _NSLOT, (b + 3) % _NSLOT, s3)

                s2 = (b + 2) % _NSLOT

                @pl.when(k < nch - 2)
                def _():
                    @pl.when(k >= 2)
                    def _():
                        wait_store(s2)

                    wait_idx(s2)
                    fire_gather(s2)

                poff = b * _CH

                @plsc.parallel_loop(0, _CH, 1, unroll=8)
                def _(l):
                    sv = jnp.full((_LANES,), l, jnp.int32)
                    for j in range(ecols):
                        sl = pl.ds(j * _LANES, _LANES)
                        v = g[b][l, sl] + pos_v[poff + l, sl]
                        plsc.store_scatter(ot[b], [etv[j], e8v[j], sv], v)

                fire_store(row, b)
            return carry

        lax.fori_loop(0, rows_per_w, group_body, 0)
        for s in range(_NSLOT):
            wait_store(s)

    return embed_kernel


def kernel(input_ids, token_table, position_table):
    batch, seq = input_ids.shape
    vocab, embed = token_table.shape
    nfull = vocab // _CH
    tailn = vocab - nfull * _CH
    tfn = _build_transpose(vocab, embed)
    lfn = _build_lookup(batch, seq, embed, vocab)
    tail128 = token_table[nfull * _CH :].reshape(tailn // 2, _CH)
    tmp = tfn(token_table.T, tail128)
    out5 = lfn(input_ids, tmp.reshape(vocab, embed), position_table)
    return out5.transpose(0, 2, 4, 1, 3).reshape(batch, seq, embed)


# speedup vs baseline: 1.2147x; 1.2147x over previous
import functools

import jax
import jax.numpy as jnp
from jax import lax
from jax.experimental import pallas as pl
from jax.experimental.pallas import tpu as pltpu
from jax.experimental.pallas import tpu_sc as plsc

_LANES = 16
_CH = 128
_NSLOT = 4
_PAD = 129


@functools.cache
def _build(batch, seq, embed, vocab):
    info = plsc.get_sparse_core_info()
    nw = info.num_cores * info.num_subcores
    assert batch % nw == 0 and seq == _NSLOT * _CH and embed == 64
    rows_per_w = batch // nw
    nch = rows_per_w * _NSLOT
    st_n = seq // _CH
    ecols = embed // _LANES

    mesh = plsc.VectorSubcoreMesh(core_axis_name="c", subcore_axis_name="s")

    @functools.partial(
        pl.kernel,
        out_type=jax.ShapeDtypeStruct((batch, 8, st_n, 8, _CH), jnp.float32),
        mesh=mesh,
        compiler_params=pltpu.CompilerParams(
            use_tc_tiling_on_sc=False, needs_layout_passes=False
        ),
        scratch_types=(
            [pltpu.VMEM((seq, embed), jnp.float32)]
            + [pltpu.VMEM((_CH,), jnp.int32) for _ in range(_NSLOT)]
            + [pltpu.VMEM((_CH, 64), jnp.float32) for _ in range(_NSLOT)]
            + [pltpu.VMEM((8, 8, _PAD), jnp.float32) for _ in range(_NSLOT)]
            + [pltpu.SemaphoreType.DMA for _ in range(3 * _NSLOT)]
        ),
    )
    def embed_kernel(ids_hbm, tok_hbm, pos_hbm, out_hbm, pos_v, *scratch):
        idx = scratch[:_NSLOT]
        g = scratch[_NSLOT : 2 * _NSLOT]
        ot = scratch[2 * _NSLOT : 3 * _NSLOT]
        sems = scratch[3 * _NSLOT :]
        isem = sems[:_NSLOT]
        gsem = sems[_NSLOT : 2 * _NSLOT]
        osem = sems[2 * _NSLOT : 3 * _NSLOT]

        wid = lax.axis_index("s") * info.num_cores + lax.axis_index("c")
        row0 = wid * rows_per_w
        pltpu.sync_copy(pos_hbm, pos_v)

        def fire_idx(row, c, s):
            pltpu.async_copy(ids_hbm.at[row, pl.ds(c * _CH, _CH)], idx[s], isem[s])

        def wait_idx(s):
            pltpu.make_async_copy(ids_hbm.at[0, pl.ds(0, _CH)], idx[s], isem[s]).wait()

        def fire_gather(s):
            pltpu.async_copy(tok_hbm.at[idx[s]], g[s], gsem[s])

        def wait_gather(s):
            pltpu.make_async_copy(tok_hbm.at[pl.ds(0, _CH)], g[s], gsem[s]).wait()

        def fire_store(row, s):
            pltpu.async_copy(
                ot[s].at[:, :, pl.ds(0, _CH)], out_hbm.at[row, :, s, :, :], osem[s]
            )

        def wait_store(s):
            pltpu.make_async_copy(
                ot[s].at[:, :, pl.ds(0, _CH)], out_hbm.at[0, :, 0, :, :], osem[s]
            ).wait()

        for s in range(3):
            fire_idx(row0, s, s)
        wait_idx(0)
        fire_gather(0)
        wait_idx(1)
        fire_gather(1)

        iota = lax.iota(jnp.int32, _LANES)
        etv = [lax.shift_right_logical(iota + 16 * j, 3) for j in range(ecols)]
        e8v = [lax.bitwise_and(iota + 16 * j, 7) for j in range(ecols)]

        def group_body(kk, carry):
            row = row0 + kk
            for b in range(_NSLOT):
                k = kk * _NSLOT + b
                wait_gather(b)

                s3 = (b + 3) % _NSLOT

                @pl.when(k < nch - 3)
                def _():
                    fire_idx(row + (b + 3) // _NSLOT, (b + 3) % _NSLOT, s3)

                s2 = (b + 2) % _NSLOT

                @pl.when(k < nch - 2)
                def _():
                    @pl.when(k >= 2)
                    def _():
                        wait_store(s2)

                    wait_idx(s2)
                    fire_gather(s2)

                poff = b * _CH

                @plsc.parallel_loop(0, _CH, 1, unroll=8)
                def _(l):
                    sv = jnp.full((_LANES,), l, jnp.int32)
                    for j in range(ecols):
                        sl = pl.ds(j * _LANES, _LANES)
                        v = g[b][l, sl] + pos_v[poff + l, sl]
                        plsc.store_scatter(ot[b], [etv[j], e8v[j], sv], v)
                fire_store(row, b)
            return carry

        lax.fori_loop(0, rows_per_w, group_body, 0)
        for s in range(_NSLOT):
            wait_store(s)

    return embed_kernel


def kernel(input_ids, token_table, position_table):
    batch, seq = input_ids.shape
    vocab, embed = token_table.shape
    fn = _build(batch, seq, embed, vocab)
    out5 = fn(input_ids, token_table, position_table)
    return out5.transpose(0, 2, 4, 1, 3).reshape(batch, seq, embed)
